# Initial kernel scaffold; baseline (speedup 1.0000x reference)
import jax
import jax.numpy as jnp
from jax.experimental import pallas as pl


def kernel(x, edge_index, W1, att_s1, att_d1, b1, W2, att_s2, att_d2, b2, lin1_W, lin1_b, lin2_W, lin2_b):
    raise NotImplementedError("write your pallas kernel here")


# speedup vs baseline: 19.7844x; 19.7844x over previous
import functools

import jax
import jax.numpy as jnp
from jax import lax
from jax.experimental import pallas as pl
from jax.experimental.pallas import tpu as pltpu
from jax.experimental.pallas import tpu_sc as plsc

N = 10000
IN = 128
HEADS = 8
DIM = 32
OUT = 16

NE = 330000
CH = 128
CHUNKS = 162
EPT = CHUNKS * CH
E_PAD = 16 * EPT
NPAD = 10240
NPT = 640
BR = 1000
NB = N // BR

_f32 = jnp.float32
_i32 = jnp.int32

_MESH = plsc.VectorSubcoreMesh(
    core_axis_name="c", subcore_axis_name="s", num_cores=2, num_subcores=16)
_SC_PARAMS = pltpu.CompilerParams(needs_layout_passes=False,
                                  use_tc_tiling_on_sc=False)


def _tc1_body(x_ref, w_ref, sa_ref, sd_ref, h_ref, as_ref, ad_ref, cs_ref,
              cd_ref):
    r = pl.program_id(1)
    h = jnp.dot(x_ref[...], w_ref[0], preferred_element_type=_f32)
    h_ref[...] = h[None]
    a_s = jnp.dot(h, sa_ref[0], preferred_element_type=_f32)
    a_d = jnp.dot(h, sd_ref[0], preferred_element_type=_f32)
    as_ref[...] = a_s[None]
    ad_ref[...] = a_d[None]

    @pl.when(r == 0)
    def _():
        cs_ref[...] = jnp.full((1, 1, 2), -jnp.inf, _f32)
        cd_ref[...] = jnp.full((1, 1, 2), -jnp.inf, _f32)

    cs_ref[...] = jnp.maximum(cs_ref[...], a_s.max(axis=0)[None, None])
    cd_ref[...] = jnp.maximum(cd_ref[...], a_d.max(axis=0)[None, None])


def _tc1(x, w1q, sa, sd):
    return pl.pallas_call(
        _tc1_body,
        grid=(4, NB),
        in_specs=[
            pl.BlockSpec((BR, IN), lambda q, r: (r, 0)),
            pl.BlockSpec((1, IN, 64), lambda q, r: (q, 0, 0)),
            pl.BlockSpec((1, 64, 2), lambda q, r: (q, 0, 0)),
            pl.BlockSpec((1, 64, 2), lambda q, r: (q, 0, 0)),
        ],
        out_specs=[
            pl.BlockSpec((1, BR, 64), lambda q, r: (q, r, 0)),
            pl.BlockSpec((1, BR, 2), lambda q, r: (q, r, 0)),
            pl.BlockSpec((1, BR, 2), lambda q, r: (q, r, 0)),
            pl.BlockSpec((1, 1, 2), lambda q, r: (q, 0, 0)),
            pl.BlockSpec((1, 1, 2), lambda q, r: (q, 0, 0)),
        ],
        out_shape=[
            jax.ShapeDtypeStruct((4, N, 64), _f32),
            jax.ShapeDtypeStruct((4, N, 2), _f32),
            jax.ShapeDtypeStruct((4, N, 2), _f32),
            jax.ShapeDtypeStruct((4, 1, 2), _f32),
            jax.ShapeDtypeStruct((4, 1, 2), _f32),
        ],
    )(x, w1q, sa, sd)


@functools.partial(
    pl.kernel,
    out_type=(
        jax.ShapeDtypeStruct((2, 2, 16, CHUNKS, 2 * CH), _f32),
        jax.ShapeDtypeStruct((2, 16, CHUNKS, 4 * CH), _f32),
    ),
    mesh=_MESH,
    compiler_params=_SC_PARAMS,
    scratch_types=[
        pltpu.VMEM((CH,), _i32),
        pltpu.VMEM((CH,), _i32),
        pltpu.VMEM((4 * CH,), _f32),
        pltpu.VMEM((4 * CH,), _f32),
        pltpu.VMEM((4 * CH,), _f32),
        pltpu.VMEM((4 * CH,), _f32),
        pltpu.VMEM((2 * CH,), _f32),
        pltpu.VMEM((2 * CH,), _f32),
        pltpu.VMEM((64,), _f32),
        [pltpu.VMEM_SHARED((NPAD,), _f32) for _ in range(4)],
        [pltpu.VMEM_SHARED((NPAD,), _f32) for _ in range(4)],
        [pltpu.VMEM_SHARED((NPAD,), _f32) for _ in range(4)],
    ],
)
def _k1a(as_hbm, ad_hbm, c1_hbm, src_hbm, dst_hbm, coef_hbm, e_hbm,
         srcc, dstc, asb, adb, ebuf, dbuf, cbufa, cbufb, c1v,
         sh_as, sh_ad, sh_den):
    c = lax.axis_index("c")
    s = lax.axis_index("s")

    pltpu.sync_copy(c1_hbm.at[c], c1v)
    for h in range(4):
        pltpu.sync_copy(as_hbm.at[c, h, pl.ds(s * NPT, NPT)],
                        sh_as[h].at[pl.ds(s * NPT, NPT)])
        pltpu.sync_copy(ad_hbm.at[c, h, pl.ds(s * NPT, NPT)],
                        sh_ad[h].at[pl.ds(s * NPT, NPT)])

    iota = lax.iota(_i32, 16)
    z16 = jnp.zeros((16,), _f32)
    for g in range(8):
        ebuf[pl.ds(16 * g, 16)] = z16
    for h in range(4):
        for i in range(5):
            pltpu.sync_copy(ebuf.at[pl.ds(0, CH)],
                            sh_den[h].at[pl.ds(s * NPT + i * CH, CH)])

    plsc.subcore_barrier()

    c1h = [c1v[pl.ds(16 * h, 16)] for h in range(4)]

    @pl.loop(0, CHUNKS)
    def _pass1(k):
        pltpu.sync_copy(src_hbm.at[s, k], srcc)
        pltpu.sync_copy(dst_hbm.at[s, k], dstc)
        for h in range(4):
            pltpu.sync_copy(sh_as[h].at[srcc], asb.at[pl.ds(CH * h, CH)])
            pltpu.sync_copy(sh_ad[h].at[dstc], adb.at[pl.ds(CH * h, CH)])
        base = s * EPT + k * CH
        for g in range(8):
            mask = (base + 16 * g + iota) < NE
            for h in range(4):
                o = CH * h + 16 * g
                a = asb[pl.ds(o, 16)] + adb[pl.ds(o, 16)]
                a = jnp.maximum(a, 0.2 * a)
                e = jnp.where(mask, jnp.exp(a - c1h[h]), 0.0)
                ebuf[pl.ds(o, 16)] = e
        pltpu.sync_copy(ebuf, e_hbm.at[c, s, k])
        for h in range(4):
            pltpu.sync_copy(ebuf.at[pl.ds(CH * h, CH)],
                            sh_den[h].at[dstc], add=True)

    plsc.subcore_barrier()

    @pl.loop(0, CHUNKS)
    def _pass2(k):
        pltpu.sync_copy(dst_hbm.at[s, k], dstc)
        pltpu.sync_copy(e_hbm.at[c, s, k], ebuf)
        for h in range(4):
            pltpu.sync_copy(sh_den[h].at[dstc], dbuf.at[pl.ds(CH * h, CH)])
        for p in range(2):
            cb = cbufa if p == 0 else cbufb
            for j in range(2):
                h = 2 * p + j
                for g in range(8):
                    o = CH * h + 16 * g
                    ev = ebuf[pl.ds(o, 16)]
                    dv = dbuf[pl.ds(o, 16)]
                    cb[pl.ds(CH * j + 16 * g, 16)] = ev / (dv + 1e-16)
        pltpu.sync_copy(cbufa, coef_hbm.at[c, 0, s, k])
        pltpu.sync_copy(cbufb, coef_hbm.at[c, 1, s, k])


def _make_k1b(p):
    @functools.partial(
        pl.kernel,
        out_type=jax.ShapeDtypeStruct((2, NPAD, 64), _f32),
        mesh=_MESH,
        compiler_params=_SC_PARAMS,
        scratch_types=[
            pltpu.VMEM((CH,), _i32),
            pltpu.VMEM((CH,), _i32),
            pltpu.VMEM((2 * CH,), _f32),
            pltpu.VMEM((CH, 64), _f32),
            pltpu.VMEM_SHARED((NPAD, 64), _f32),
        ],
    )
    def _k1b(h_hbm, coef_hbm, adj_hbm, dst_hbm, out_hbm,
             adjc, dstc, cbuf, hbuf, sh_out):
        c = lax.axis_index("c")
        s = lax.axis_index("s")

        z16 = jnp.zeros((16,), _f32)

        @pl.loop(0, CH)
        def _zrow(r):
            for u in range(4):
                hbuf[r, pl.ds(16 * u, 16)] = z16

        for i in range(5):
            pltpu.sync_copy(hbuf, sh_out.at[pl.ds(s * NPT + i * CH, CH)])

        plsc.subcore_barrier()

        @pl.loop(0, CHUNKS)
        def _msg(k):
            pltpu.sync_copy(adj_hbm.at[c, s, k], adjc)
            pltpu.sync_copy(dst_hbm.at[s, k], dstc)
            pltpu.sync_copy(coef_hbm.at[c, p, s, k], cbuf)
            pltpu.sync_copy(h_hbm.at[adjc], hbuf)

            @pl.loop(0, CH)
            def _mul(e):
                for j in range(2):
                    cs = plsc.load_gather(
                        cbuf, [jnp.full((16,), CH * j + e, _i32)])
                    for u in range(2):
                        off = 32 * j + 16 * u
                        hbuf[e, pl.ds(off, 16)] = hbuf[e, pl.ds(off, 16)] * cs

            pltpu.sync_copy(hbuf, sh_out.at[dstc], add=True)

        plsc.subcore_barrier()
        for i in range(5):
            pltpu.sync_copy(sh_out.at[pl.ds(s * NPT + i * CH, CH)], hbuf)
            pltpu.sync_copy(hbuf, out_hbm.at[c, pl.ds(s * NPT + i * CH, CH)])

    return _k1b


_K1B = (_make_k1b(0), _make_k1b(1))


def _tc2_body(o1a_ref, o1b_ref, b1_ref, w2_ref, sd2_ref, h2_ref, asd_ref,
              cm_ref):
    r = pl.program_id(0)
    h2 = jnp.zeros((BR, DIM), _f32)
    for i, (ref, ci) in enumerate(
            ((o1a_ref, 0), (o1b_ref, 0), (o1a_ref, 1), (o1b_ref, 1))):
        x1 = ref[ci] + b1_ref[i][None]
        x1 = jnp.where(x1 > 0, x1, jnp.exp(jnp.minimum(x1, 0.0)) - 1.0)
        h2 = h2 + jnp.dot(x1, w2_ref[i], preferred_element_type=_f32)
    h2_ref[...] = h2
    asd = jnp.dot(h2, sd2_ref[...], preferred_element_type=_f32)
    asd_ref[...] = asd

    @pl.when(r == 0)
    def _():
        cm_ref[...] = jnp.full((1, 1, 2), -jnp.inf, _f32)

    cm_ref[...] = jnp.maximum(cm_ref[...], asd.max(axis=0)[None, None])


def _tc2(o1a, o1b, b1q, w2q, sd2):
    return pl.pallas_call(
        _tc2_body,
        grid=(NB,),
        in_specs=[
            pl.BlockSpec((2, BR, 64), lambda r: (0, r, 0)),
            pl.BlockSpec((2, BR, 64), lambda r: (0, r, 0)),
            pl.BlockSpec((4, 64), lambda r: (0, 0)),
            pl.BlockSpec((4, 64, DIM), lambda r: (0, 0, 0)),
            pl.BlockSpec((DIM, 2), lambda r: (0, 0)),
        ],
        out_specs=[
            pl.BlockSpec((BR, DIM), lambda r: (r, 0)),
            pl.BlockSpec((BR, 2), lambda r: (r, 0)),
            pl.BlockSpec((1, 1, 2), lambda r: (0, 0, 0)),
        ],
        out_shape=[
            jax.ShapeDtypeStruct((N, DIM), _f32),
            jax.ShapeDtypeStruct((N, 2), _f32),
            jax.ShapeDtypeStruct((1, 1, 2), _f32),
        ],
    )(o1a, o1b, b1q, w2q, sd2)


@functools.partial(
    pl.kernel,
    out_type=jax.ShapeDtypeStruct((NPAD,), _f32),
    mesh=_MESH,
    compiler_params=_SC_PARAMS,
    scratch_types=[
        pltpu.VMEM((N,), _f32),
        pltpu.VMEM((N,), _f32),
        pltpu.VMEM((NPAD,), _f32),
        pltpu.VMEM((CHUNKS, CH), _f32),
        pltpu.VMEM((CH,), _i32),
        pltpu.VMEM((CH,), _i32),
        pltpu.VMEM((CH,), _f32),
        pltpu.VMEM((16,), _f32),
        pltpu.VMEM_SHARED((NPAD,), _f32),
        pltpu.VMEM_SHARED((NPAD,), _f32),
    ],
)
def _k2(as2_hbm, ad2_hbm, c2_hbm, src_hbm, dst_hbm, c_hbm,
        as2fl, ad2fl, d2b, e2buf, srcc, dstc, cbuf, c2v, sh_d2, sh_c):
    c = lax.axis_index("c")
    s = lax.axis_index("s")

    @pl.when(c == 0)
    def _():
        pltpu.sync_copy(as2_hbm, as2fl)
        pltpu.sync_copy(ad2_hbm, ad2fl)
        pltpu.sync_copy(c2_hbm, c2v)

        z16 = jnp.zeros((16,), _f32)

        @pl.loop(0, CH // 16)
        def _zr1(r):
            d2b[pl.ds(16 * r, 16)] = z16

        for i in range(5):
            pltpu.sync_copy(d2b.at[pl.ds(0, CH)],
                            sh_d2.at[pl.ds(s * NPT + i * CH, CH)])
            pltpu.sync_copy(d2b.at[pl.ds(0, CH)],
                            sh_c.at[pl.ds(s * NPT + i * CH, CH)])

        plsc.subcore_barrier()

        iota = lax.iota(_i32, 16)
        c2t = c2v[...]

        @pl.loop(0, CHUNKS)
        def _pass1(k):
            pltpu.sync_copy(src_hbm.at[s, k], srcc)
            pltpu.sync_copy(dst_hbm.at[s, k], dstc)
            base = s * EPT + k * CH
            for g in range(8):
                sv = srcc[pl.ds(16 * g, 16)]
                dv = dstc[pl.ds(16 * g, 16)]
                a = (plsc.load_gather(as2fl, [sv])
                     + plsc.load_gather(ad2fl, [dv]))
                a = jnp.maximum(a, 0.2 * a)
                e = jnp.where((base + 16 * g + iota) < NE,
                              jnp.exp(a - c2t), 0.0)
                e2buf[k, pl.ds(16 * g, 16)] = e
            pltpu.sync_copy(e2buf.at[k], sh_d2.at[dstc], add=True)

        plsc.subcore_barrier()
        pltpu.sync_copy(sh_d2, d2b)

        @pl.loop(0, CHUNKS)
        def _pass2(k):
            pltpu.sync_copy(src_hbm.at[s, k], srcc)
            pltpu.sync_copy(dst_hbm.at[s, k], dstc)
            for g in range(8):
                ev = e2buf[k, pl.ds(16 * g, 16)]
                dv2 = plsc.load_gather(d2b, [dstc[pl.ds(16 * g, 16)]])
                cbuf[pl.ds(16 * g, 16)] = ev / (dv2 + 1e-16)
            pltpu.sync_copy(cbuf, sh_c.at[srcc], add=True)

        plsc.subcore_barrier()
        pltpu.sync_copy(sh_c.at[pl.ds(s * NPT, NPT)],
                        c_hbm.at[pl.ds(s * NPT, NPT)])


def _tc3_body(c_ref, h2_ref, b2_ref, l1w_ref, l1b_ref, l2w_ref, l2b_ref,
              out_ref, acc_ref):
    r = pl.program_id(0)

    @pl.when(r == 0)
    def _():
        acc_ref[...] = jnp.zeros((1, DIM), _f32)

    acc_ref[...] += jnp.dot(c_ref[0], h2_ref[...],
                            preferred_element_type=_f32)

    @pl.when(r == NB - 1)
    def _():
        g = acc_ref[...] + float(N) * b2_ref[...]
        g1 = jnp.maximum(
            jnp.dot(g, l1w_ref[...], preferred_element_type=_f32)
            + l1b_ref[...], 0.0)
        out_ref[...] = (jnp.dot(g1, l2w_ref[...], preferred_element_type=_f32)
                        + l2b_ref[...])


def _tc3(c2d, h2, b2r, l1w, l1b, l2w, l2b):
    return pl.pallas_call(
        _tc3_body,
        grid=(NB,),
        in_specs=[
            pl.BlockSpec((1, 1, BR), lambda r: (r, 0, 0)),
            pl.BlockSpec((BR, DIM), lambda r: (r, 0)),
            pl.BlockSpec((1, DIM), lambda r: (0, 0)),
            pl.BlockSpec((DIM, DIM), lambda r: (0, 0)),
            pl.BlockSpec((1, DIM), lambda r: (0, 0)),
            pl.BlockSpec((DIM, OUT), lambda r: (0, 0)),
            pl.BlockSpec((1, OUT), lambda r: (0, 0)),
        ],
        out_specs=pl.BlockSpec((1, OUT), lambda r: (0, 0)),
        out_shape=jax.ShapeDtypeStruct((1, OUT), _f32),
        scratch_shapes=[pltpu.VMEM((1, DIM), _f32)],
    )(c2d, h2, b2r, l1w, l1b, l2w, l2b)


def _blockdiag2(att):
    z = jnp.zeros((2, DIM, 2), att.dtype)
    z = z.at[jnp.arange(2), :, jnp.arange(2)].set(att)
    return z.reshape(2 * DIM, 2)


def kernel(x, edge_index, W1, att_s1, att_d1, b1, W2, att_s2, att_d2, b2,
           lin1_W, lin1_b, lin2_W, lin2_b):
    loops = jnp.arange(N, dtype=_i32)
    pad = jnp.zeros((E_PAD - NE,), _i32)
    src = jnp.concatenate([edge_index[0].astype(_i32), loops, pad])
    dst = jnp.concatenate([edge_index[1].astype(_i32), loops, pad])
    src3 = src.reshape(16, CHUNKS, CH)
    dst3 = dst.reshape(16, CHUNKS, CH)
    adj4 = src3[None] + (jnp.arange(4, dtype=_i32) * N)[:, None, None, None]

    w1q = W1.reshape(IN, 4, 64).transpose(1, 0, 2)
    sa = jnp.stack([_blockdiag2(att_s1[2 * q:2 * q + 2]) for q in range(4)])
    sd = jnp.stack([_blockdiag2(att_d1[2 * q:2 * q + 2]) for q in range(4)])

    h4, as4, ad4, cs, cd = _tc1(x.astype(_f32), w1q, sa, sd)
    as_p = jnp.pad(as4.transpose(0, 2, 1).reshape(2, 4, N),
                   ((0, 0), (0, 0), (0, NPAD - N)))
    ad_p = jnp.pad(ad4.transpose(0, 2, 1).reshape(2, 4, N),
                   ((0, 0), (0, 0), (0, NPAD - N)))
    c1 = (cs + cd).reshape(2, 4)
    c1 = jnp.tile(c1[:, :, None], (1, 1, 16)).reshape(2, 64)

    coef, _unused_e = _k1a(as_p, ad_p, c1, src3, dst3)

    h_flat = h4.reshape(4 * N, 64)
    o1a = _K1B[0](h_flat, coef, adj4[0::2], dst3)
    o1b = _K1B[1](h_flat, coef, adj4[1::2], dst3)

    sd2 = jnp.stack([att_s2[0], att_d2[0]], axis=1)
    b1q = b1.reshape(4, 64)
    w2q = W2.reshape(4, 64, DIM)
    h2, asd2, cm2 = _tc2(o1a, o1b, b1q, w2q, sd2)
    c2arr = jnp.full((16,), cm2[0, 0, 0] + cm2[0, 0, 1], _f32)

    c_pad = _k2(asd2[:, 0], asd2[:, 1], c2arr, src3, dst3)

    c2d = c_pad[:N].reshape(NB, 1, BR)
    return _tc3(c2d, h2, b2.reshape(1, DIM), lin1_W, lin1_b.reshape(1, DIM),
                lin2_W, lin2_b.reshape(1, OUT))


# speedup vs baseline: 23.3164x; 1.1785x over previous
import functools

import jax
import jax.numpy as jnp
from jax import lax
from jax.experimental import pallas as pl
from jax.experimental.pallas import tpu as pltpu
from jax.experimental.pallas import tpu_sc as plsc

N = 10000
IN = 128
HEADS = 8
DIM = 32
OUT = 16

NE = 330000
CH = 128
CHUNKS = 162
EPT = CHUNKS * CH
E_PAD = 16 * EPT
NPAD = 10240
NPT = 640
BR = 1000
NB = N // BR

_f32 = jnp.float32
_i32 = jnp.int32

_MESH = plsc.VectorSubcoreMesh(
    core_axis_name="c", subcore_axis_name="s", num_cores=2, num_subcores=16)
_SC_PARAMS = pltpu.CompilerParams(needs_layout_passes=False,
                                  use_tc_tiling_on_sc=False)


def _tc1_body(x_ref, w_ref, sa_ref, sd_ref, h_ref, as_ref, ad_ref, cs_ref,
              cd_ref):
    r = pl.program_id(1)
    h = jnp.dot(x_ref[...], w_ref[0], preferred_element_type=_f32)
    h_ref[...] = h[None]
    a_s = jnp.dot(h, sa_ref[0], preferred_element_type=_f32)
    a_d = jnp.dot(h, sd_ref[0], preferred_element_type=_f32)
    as_ref[...] = a_s[None]
    ad_ref[...] = a_d[None]

    @pl.when(r == 0)
    def _():
        cs_ref[...] = jnp.full((1, 1, 2), -jnp.inf, _f32)
        cd_ref[...] = jnp.full((1, 1, 2), -jnp.inf, _f32)

    cs_ref[...] = jnp.maximum(cs_ref[...], a_s.max(axis=0)[None, None])
    cd_ref[...] = jnp.maximum(cd_ref[...], a_d.max(axis=0)[None, None])


def _tc1(x, w1q, sa, sd):
    return pl.pallas_call(
        _tc1_body,
        grid=(4, NB),
        in_specs=[
            pl.BlockSpec((BR, IN), lambda q, r: (r, 0)),
            pl.BlockSpec((1, IN, 64), lambda q, r: (q, 0, 0)),
            pl.BlockSpec((1, 64, 2), lambda q, r: (q, 0, 0)),
            pl.BlockSpec((1, 64, 2), lambda q, r: (q, 0, 0)),
        ],
        out_specs=[
            pl.BlockSpec((1, BR, 64), lambda q, r: (q, r, 0)),
            pl.BlockSpec((1, BR, 2), lambda q, r: (q, r, 0)),
            pl.BlockSpec((1, BR, 2), lambda q, r: (q, r, 0)),
            pl.BlockSpec((1, 1, 2), lambda q, r: (q, 0, 0)),
            pl.BlockSpec((1, 1, 2), lambda q, r: (q, 0, 0)),
        ],
        out_shape=[
            jax.ShapeDtypeStruct((4, N, 64), _f32),
            jax.ShapeDtypeStruct((4, N, 2), _f32),
            jax.ShapeDtypeStruct((4, N, 2), _f32),
            jax.ShapeDtypeStruct((4, 1, 2), _f32),
            jax.ShapeDtypeStruct((4, 1, 2), _f32),
        ],
    )(x, w1q, sa, sd)


@functools.partial(
    pl.kernel,
    out_type=(
        jax.ShapeDtypeStruct((2, 2, 16, CHUNKS, 2 * CH), _f32),
        jax.ShapeDtypeStruct((2, 16, CHUNKS, 4 * CH), _f32),
    ),
    mesh=_MESH,
    compiler_params=_SC_PARAMS,
    scratch_types=[
        pltpu.VMEM((CH,), _i32),
        pltpu.VMEM((CH,), _i32),
        pltpu.VMEM((4 * CH,), _f32),
        pltpu.VMEM((4 * CH,), _f32),
        pltpu.VMEM((4 * CH,), _f32),
        pltpu.VMEM((4 * CH,), _f32),
        pltpu.VMEM((2 * CH,), _f32),
        pltpu.VMEM((2 * CH,), _f32),
        pltpu.VMEM((64,), _f32),
        [pltpu.VMEM_SHARED((NPAD,), _f32) for _ in range(4)],
        [pltpu.VMEM_SHARED((NPAD,), _f32) for _ in range(4)],
        [pltpu.VMEM_SHARED((NPAD,), _f32) for _ in range(4)],
    ],
)
def _k1a(as_hbm, ad_hbm, c1_hbm, src_hbm, dst_hbm, coef_hbm, e_hbm,
         srcc, dstc, asb, adb, ebuf, dbuf, cbufa, cbufb, c1v,
         sh_as, sh_ad, sh_den):
    c = lax.axis_index("c")
    s = lax.axis_index("s")

    pltpu.sync_copy(c1_hbm.at[c], c1v)
    for h in range(4):
        pltpu.sync_copy(as_hbm.at[c, h, pl.ds(s * NPT, NPT)],
                        sh_as[h].at[pl.ds(s * NPT, NPT)])
        pltpu.sync_copy(ad_hbm.at[c, h, pl.ds(s * NPT, NPT)],
                        sh_ad[h].at[pl.ds(s * NPT, NPT)])

    iota = lax.iota(_i32, 16)
    z16 = jnp.zeros((16,), _f32)
    for g in range(8):
        ebuf[pl.ds(16 * g, 16)] = z16
    for h in range(4):
        for i in range(5):
            pltpu.sync_copy(ebuf.at[pl.ds(0, CH)],
                            sh_den[h].at[pl.ds(s * NPT + i * CH, CH)])

    plsc.subcore_barrier()

    c1h = [c1v[pl.ds(16 * h, 16)] for h in range(4)]

    @pl.loop(0, CHUNKS)
    def _pass1(k):
        pltpu.sync_copy(src_hbm.at[s, k], srcc)
        pltpu.sync_copy(dst_hbm.at[s, k], dstc)
        for h in range(4):
            pltpu.sync_copy(sh_as[h].at[srcc], asb.at[pl.ds(CH * h, CH)])
            pltpu.sync_copy(sh_ad[h].at[dstc], adb.at[pl.ds(CH * h, CH)])
        base = s * EPT + k * CH
        for g in range(8):
            mask = (base + 16 * g + iota) < NE
            for h in range(4):
                o = CH * h + 16 * g
                a = asb[pl.ds(o, 16)] + adb[pl.ds(o, 16)]
                a = jnp.maximum(a, 0.2 * a)
                e = jnp.where(mask, jnp.exp(a - c1h[h]), 0.0)
                ebuf[pl.ds(o, 16)] = e
        pltpu.sync_copy(ebuf, e_hbm.at[c, s, k])
        for h in range(4):
            pltpu.sync_copy(ebuf.at[pl.ds(CH * h, CH)],
                            sh_den[h].at[dstc], add=True)

    plsc.subcore_barrier()

    @pl.loop(0, CHUNKS)
    def _pass2(k):
        pltpu.sync_copy(dst_hbm.at[s, k], dstc)
        pltpu.sync_copy(e_hbm.at[c, s, k], ebuf)
        for h in range(4):
            pltpu.sync_copy(sh_den[h].at[dstc], dbuf.at[pl.ds(CH * h, CH)])
        for p in range(2):
            cb = cbufa if p == 0 else cbufb
            for j in range(2):
                h = 2 * p + j
                for g in range(8):
                    o = CH * h + 16 * g
                    ev = ebuf[pl.ds(o, 16)]
                    dv = dbuf[pl.ds(o, 16)]
                    cb[pl.ds(CH * j + 16 * g, 16)] = ev / (dv + 1e-16)
        pltpu.sync_copy(cbufa, coef_hbm.at[c, 0, s, k])
        pltpu.sync_copy(cbufb, coef_hbm.at[c, 1, s, k])


def _make_k1b(p):
    @functools.partial(
        pl.kernel,
        out_type=jax.ShapeDtypeStruct((2, NPAD, 64), _f32),
        mesh=_MESH,
        compiler_params=_SC_PARAMS,
        scratch_types=[
            pltpu.VMEM((CH,), _i32),
            pltpu.VMEM((CH,), _i32),
            pltpu.VMEM((CH,), _i32),
            pltpu.VMEM((CH,), _i32),
            pltpu.VMEM((2 * CH,), _f32),
            pltpu.VMEM((2 * CH,), _f32),
            pltpu.VMEM((CH, 64), _f32),
            pltpu.VMEM((CH, 64), _f32),
            pltpu.SemaphoreType.DMA,
            pltpu.SemaphoreType.DMA,
            pltpu.SemaphoreType.DMA,
            pltpu.SemaphoreType.DMA,
            pltpu.VMEM_SHARED((NPAD, 64), _f32),
        ],
    )
    def _k1b(h_hbm, coef_hbm, adj_hbm, dst_hbm, out_hbm,
             adjc0, adjc1, dstc0, dstc1, cbuf0, cbuf1, hbuf0, hbuf1,
             gsem0, gsem1, ssem0, ssem1, sh_out):
        c = lax.axis_index("c")
        s = lax.axis_index("s")
        adjc = (adjc0, adjc1)
        dstc = (dstc0, dstc1)
        cbuf = (cbuf0, cbuf1)
        hbuf = (hbuf0, hbuf1)
        gsem = (gsem0, gsem1)
        ssem = (ssem0, ssem1)

        z16 = jnp.zeros((16,), _f32)

        @pl.loop(0, CH)
        def _zrow(r):
            for u in range(4):
                hbuf0[r, pl.ds(16 * u, 16)] = z16

        for i in range(5):
            pltpu.sync_copy(hbuf0, sh_out.at[pl.ds(s * NPT + i * CH, CH)])

        plsc.subcore_barrier()

        def load_and_gather(b, k):
            pltpu.sync_copy(adj_hbm.at[c, s, k], adjc[b])
            pltpu.sync_copy(dst_hbm.at[s, k], dstc[b])
            pltpu.sync_copy(coef_hbm.at[c, p, s, k], cbuf[b])
            pltpu.async_copy(h_hbm.at[adjc[b]], hbuf[b], gsem[b])

        def mult(b):
            @pl.loop(0, CH)
            def _mul(e):
                for j in range(2):
                    cs = plsc.load_gather(
                        cbuf[b], [jnp.full((16,), CH * j + e, _i32)])
                    for u in range(2):
                        off = 32 * j + 16 * u
                        hbuf[b][e, pl.ds(off, 16)] = (
                            hbuf[b][e, pl.ds(off, 16)] * cs)

        load_and_gather(0, 0)
        load_and_gather(1, 1)

        NH = CHUNKS // 2

        @pl.loop(0, NH)
        def _msg(t):
            for b in range(2):
                pltpu.make_async_copy(h_hbm.at[adjc[b]], hbuf[b],
                                      gsem[b]).wait()
                mult(b)
                pltpu.async_copy(hbuf[b], sh_out.at[dstc[b]], ssem[b],
                                 add=True)

            @pl.when(t + 1 < NH)
            def _():
                for b in range(2):
                    pltpu.make_async_copy(hbuf[b], sh_out.at[dstc[b]],
                                          ssem[b]).wait()
                    load_and_gather(b, 2 * t + 2 + b)

        for b in range(2):
            pltpu.make_async_copy(hbuf[b], sh_out.at[dstc[b]],
                                  ssem[b]).wait()

        plsc.subcore_barrier()
        for i in range(5):
            pltpu.sync_copy(sh_out.at[pl.ds(s * NPT + i * CH, CH)], hbuf0)
            pltpu.sync_copy(hbuf0, out_hbm.at[c, pl.ds(s * NPT + i * CH, CH)])

    return _k1b


_K1B = (_make_k1b(0), _make_k1b(1))


def _tc2_body(o1a_ref, o1b_ref, b1_ref, w2_ref, sd2_ref, h2_ref, asd_ref,
              cm_ref):
    r = pl.program_id(0)
    h2 = jnp.zeros((BR, DIM), _f32)
    for i, (ref, ci) in enumerate(
            ((o1a_ref, 0), (o1b_ref, 0), (o1a_ref, 1), (o1b_ref, 1))):
        x1 = ref[ci] + b1_ref[i][None]
        x1 = jnp.where(x1 > 0, x1, jnp.exp(jnp.minimum(x1, 0.0)) - 1.0)
        h2 = h2 + jnp.dot(x1, w2_ref[i], preferred_element_type=_f32)
    h2_ref[...] = h2
    asd = jnp.dot(h2, sd2_ref[...], preferred_element_type=_f32)
    asd_ref[...] = asd

    @pl.when(r == 0)
    def _():
        cm_ref[...] = jnp.full((1, 1, 2), -jnp.inf, _f32)

    cm_ref[...] = jnp.maximum(cm_ref[...], asd.max(axis=0)[None, None])


def _tc2(o1a, o1b, b1q, w2q, sd2):
    return pl.pallas_call(
        _tc2_body,
        grid=(NB,),
        in_specs=[
            pl.BlockSpec((2, BR, 64), lambda r: (0, r, 0)),
            pl.BlockSpec((2, BR, 64), lambda r: (0, r, 0)),
            pl.BlockSpec((4, 64), lambda r: (0, 0)),
            pl.BlockSpec((4, 64, DIM), lambda r: (0, 0, 0)),
            pl.BlockSpec((DIM, 2), lambda r: (0, 0)),
        ],
        out_specs=[
            pl.BlockSpec((BR, DIM), lambda r: (r, 0)),
            pl.BlockSpec((BR, 2), lambda r: (r, 0)),
            pl.BlockSpec((1, 1, 2), lambda r: (0, 0, 0)),
        ],
        out_shape=[
            jax.ShapeDtypeStruct((N, DIM), _f32),
            jax.ShapeDtypeStruct((N, 2), _f32),
            jax.ShapeDtypeStruct((1, 1, 2), _f32),
        ],
    )(o1a, o1b, b1q, w2q, sd2)


@functools.partial(
    pl.kernel,
    out_type=jax.ShapeDtypeStruct((NPAD,), _f32),
    mesh=_MESH,
    compiler_params=_SC_PARAMS,
    scratch_types=[
        pltpu.VMEM((N,), _f32),
        pltpu.VMEM((N,), _f32),
        pltpu.VMEM((NPAD,), _f32),
        pltpu.VMEM((CHUNKS, CH), _f32),
        pltpu.VMEM((CH,), _i32),
        pltpu.VMEM((CH,), _i32),
        pltpu.VMEM((CH,), _f32),
        pltpu.VMEM((16,), _f32),
        pltpu.VMEM_SHARED((NPAD,), _f32),
        pltpu.VMEM_SHARED((NPAD,), _f32),
    ],
)
def _k2(as2_hbm, ad2_hbm, c2_hbm, src_hbm, dst_hbm, c_hbm,
        as2fl, ad2fl, d2b, e2buf, srcc, dstc, cbuf, c2v, sh_d2, sh_c):
    c = lax.axis_index("c")
    s = lax.axis_index("s")

    @pl.when(c == 0)
    def _():
        pltpu.sync_copy(as2_hbm, as2fl)
        pltpu.sync_copy(ad2_hbm, ad2fl)
        pltpu.sync_copy(c2_hbm, c2v)

        z16 = jnp.zeros((16,), _f32)

        @pl.loop(0, CH // 16)
        def _zr1(r):
            d2b[pl.ds(16 * r, 16)] = z16

        for i in range(5):
            pltpu.sync_copy(d2b.at[pl.ds(0, CH)],
                            sh_d2.at[pl.ds(s * NPT + i * CH, CH)])
            pltpu.sync_copy(d2b.at[pl.ds(0, CH)],
                            sh_c.at[pl.ds(s * NPT + i * CH, CH)])

        plsc.subcore_barrier()

        iota = lax.iota(_i32, 16)
        c2t = c2v[...]

        @pl.loop(0, CHUNKS)
        def _pass1(k):
            pltpu.sync_copy(src_hbm.at[s, k], srcc)
            pltpu.sync_copy(dst_hbm.at[s, k], dstc)
            base = s * EPT + k * CH
            for g in range(8):
                sv = srcc[pl.ds(16 * g, 16)]
                dv = dstc[pl.ds(16 * g, 16)]
                a = (plsc.load_gather(as2fl, [sv])
                     + plsc.load_gather(ad2fl, [dv]))
                a = jnp.maximum(a, 0.2 * a)
                e = jnp.where((base + 16 * g + iota) < NE,
                              jnp.exp(a - c2t), 0.0)
                e2buf[k, pl.ds(16 * g, 16)] = e
            pltpu.sync_copy(e2buf.at[k], sh_d2.at[dstc], add=True)

        plsc.subcore_barrier()
        pltpu.sync_copy(sh_d2, d2b)

        @pl.loop(0, CHUNKS)
        def _pass2(k):
            pltpu.sync_copy(src_hbm.at[s, k], srcc)
            pltpu.sync_copy(dst_hbm.at[s, k], dstc)
            for g in range(8):
                ev = e2buf[k, pl.ds(16 * g, 16)]
                dv2 = plsc.load_gather(d2b, [dstc[pl.ds(16 * g, 16)]])
                cbuf[pl.ds(16 * g, 16)] = ev / (dv2 + 1e-16)
            pltpu.sync_copy(cbuf, sh_c.at[srcc], add=True)

        plsc.subcore_barrier()
        pltpu.sync_copy(sh_c.at[pl.ds(s * NPT, NPT)],
                        c_hbm.at[pl.ds(s * NPT, NPT)])


def _tc3_body(c_ref, h2_ref, b2_ref, l1w_ref, l1b_ref, l2w_ref, l2b_ref,
              out_ref, acc_ref):
    r = pl.program_id(0)

    @pl.when(r == 0)
    def _():
        acc_ref[...] = jnp.zeros((1, DIM), _f32)

    acc_ref[...] += jnp.dot(c_ref[0], h2_ref[...],
                            preferred_element_type=_f32)

    @pl.when(r == NB - 1)
    def _():
        g = acc_ref[...] + float(N) * b2_ref[...]
        g1 = jnp.maximum(
            jnp.dot(g, l1w_ref[...], preferred_element_type=_f32)
            + l1b_ref[...], 0.0)
        out_ref[...] = (jnp.dot(g1, l2w_ref[...], preferred_element_type=_f32)
                        + l2b_ref[...])


def _tc3(c2d, h2, b2r, l1w, l1b, l2w, l2b):
    return pl.pallas_call(
        _tc3_body,
        grid=(NB,),
        in_specs=[
            pl.BlockSpec((1, 1, BR), lambda r: (r, 0, 0)),
            pl.BlockSpec((BR, DIM), lambda r: (r, 0)),
            pl.BlockSpec((1, DIM), lambda r: (0, 0)),
            pl.BlockSpec((DIM, DIM), lambda r: (0, 0)),
            pl.BlockSpec((1, DIM), lambda r: (0, 0)),
            pl.BlockSpec((DIM, OUT), lambda r: (0, 0)),
            pl.BlockSpec((1, OUT), lambda r: (0, 0)),
        ],
        out_specs=pl.BlockSpec((1, OUT), lambda r: (0, 0)),
        out_shape=jax.ShapeDtypeStruct((1, OUT), _f32),
        scratch_shapes=[pltpu.VMEM((1, DIM), _f32)],
    )(c2d, h2, b2r, l1w, l1b, l2w, l2b)


def _blockdiag2(att):
    z = jnp.zeros((2, DIM, 2), att.dtype)
    z = z.at[jnp.arange(2), :, jnp.arange(2)].set(att)
    return z.reshape(2 * DIM, 2)


def kernel(x, edge_index, W1, att_s1, att_d1, b1, W2, att_s2, att_d2, b2,
           lin1_W, lin1_b, lin2_W, lin2_b):
    loops = jnp.arange(N, dtype=_i32)
    pad = jnp.zeros((E_PAD - NE,), _i32)
    src = jnp.concatenate([edge_index[0].astype(_i32), loops, pad])
    dst = jnp.concatenate([edge_index[1].astype(_i32), loops, pad])
    src3 = src.reshape(16, CHUNKS, CH)
    dst3 = dst.reshape(16, CHUNKS, CH)
    adj4 = src3[None] + (jnp.arange(4, dtype=_i32) * N)[:, None, None, None]

    w1q = W1.reshape(IN, 4, 64).transpose(1, 0, 2)
    sa = jnp.stack([_blockdiag2(att_s1[2 * q:2 * q + 2]) for q in range(4)])
    sd = jnp.stack([_blockdiag2(att_d1[2 * q:2 * q + 2]) for q in range(4)])

    h4, as4, ad4, cs, cd = _tc1(x.astype(_f32), w1q, sa, sd)
    as_p = jnp.pad(as4.transpose(0, 2, 1).reshape(2, 4, N),
                   ((0, 0), (0, 0), (0, NPAD - N)))
    ad_p = jnp.pad(ad4.transpose(0, 2, 1).reshape(2, 4, N),
                   ((0, 0), (0, 0), (0, NPAD - N)))
    c1 = (cs + cd).reshape(2, 4)
    c1 = jnp.tile(c1[:, :, None], (1, 1, 16)).reshape(2, 64)

    coef, _unused_e = _k1a(as_p, ad_p, c1, src3, dst3)

    h_flat = h4.reshape(4 * N, 64)
    o1a = _K1B[0](h_flat, coef, adj4[0::2], dst3)
    o1b = _K1B[1](h_flat, coef, adj4[1::2], dst3)

    sd2 = jnp.stack([att_s2[0], att_d2[0]], axis=1)
    b1q = b1.reshape(4, 64)
    w2q = W2.reshape(4, 64, DIM)
    h2, asd2, cm2 = _tc2(o1a, o1b, b1q, w2q, sd2)
    c2arr = jnp.full((16,), cm2[0, 0, 0] + cm2[0, 0, 1], _f32)

    c_pad = _k2(asd2[:, 0], asd2[:, 1], c2arr, src3, dst3)

    c2d = c_pad[:N].reshape(NB, 1, BR)
    return _tc3(c2d, h2, b2.reshape(1, DIM), lin1_W, lin1_b.reshape(1, DIM),
                lin2_W, lin2_b.reshape(1, OUT))


# speedup vs baseline: 27.2654x; 1.1694x over previous
import functools

import jax
import jax.numpy as jnp
from jax import lax
from jax.experimental import pallas as pl
from jax.experimental.pallas import tpu as pltpu
from jax.experimental.pallas import tpu_sc as plsc

N = 10000
IN = 128
HEADS = 8
DIM = 32
OUT = 16

NE = 330000
CH = 128
CHUNKS = 162
EPT = CHUNKS * CH
E_PAD = 16 * EPT
NPAD = 10240
NPT = 640
BR = 1000
NB = N // BR

_f32 = jnp.float32
_i32 = jnp.int32

_MESH = plsc.VectorSubcoreMesh(
    core_axis_name="c", subcore_axis_name="s", num_cores=2, num_subcores=16)
_SC_PARAMS = pltpu.CompilerParams(needs_layout_passes=False,
                                  use_tc_tiling_on_sc=False)


def _tc1_body(x_ref, w_ref, sa_ref, sd_ref, h_ref, as_ref, ad_ref, cs_ref,
              cd_ref):
    r = pl.program_id(1)
    h = jnp.dot(x_ref[...], w_ref[0], preferred_element_type=_f32)
    h_ref[...] = h[None]
    a_s = jnp.dot(h, sa_ref[0], preferred_element_type=_f32)
    a_d = jnp.dot(h, sd_ref[0], preferred_element_type=_f32)
    as_ref[...] = a_s[None]
    ad_ref[...] = a_d[None]

    @pl.when(r == 0)
    def _():
        cs_ref[...] = jnp.full((1, 1, 2), -jnp.inf, _f32)
        cd_ref[...] = jnp.full((1, 1, 2), -jnp.inf, _f32)

    cs_ref[...] = jnp.maximum(cs_ref[...], a_s.max(axis=0)[None, None])
    cd_ref[...] = jnp.maximum(cd_ref[...], a_d.max(axis=0)[None, None])


def _tc1(x, w1q, sa, sd):
    return pl.pallas_call(
        _tc1_body,
        grid=(4, NB),
        in_specs=[
            pl.BlockSpec((BR, IN), lambda q, r: (r, 0)),
            pl.BlockSpec((1, IN, 64), lambda q, r: (q, 0, 0)),
            pl.BlockSpec((1, 64, 2), lambda q, r: (q, 0, 0)),
            pl.BlockSpec((1, 64, 2), lambda q, r: (q, 0, 0)),
        ],
        out_specs=[
            pl.BlockSpec((1, BR, 64), lambda q, r: (q, r, 0)),
            pl.BlockSpec((1, BR, 2), lambda q, r: (q, r, 0)),
            pl.BlockSpec((1, BR, 2), lambda q, r: (q, r, 0)),
            pl.BlockSpec((1, 1, 2), lambda q, r: (q, 0, 0)),
            pl.BlockSpec((1, 1, 2), lambda q, r: (q, 0, 0)),
        ],
        out_shape=[
            jax.ShapeDtypeStruct((4, N, 64), _f32),
            jax.ShapeDtypeStruct((4, N, 2), _f32),
            jax.ShapeDtypeStruct((4, N, 2), _f32),
            jax.ShapeDtypeStruct((4, 1, 2), _f32),
            jax.ShapeDtypeStruct((4, 1, 2), _f32),
        ],
    )(x, w1q, sa, sd)


@functools.partial(
    pl.kernel,
    out_type=(
        jax.ShapeDtypeStruct((2, 2, 16, CHUNKS, 2 * CH), _f32),
        jax.ShapeDtypeStruct((2, 16, CHUNKS, 4 * CH), _f32),
    ),
    mesh=_MESH,
    compiler_params=_SC_PARAMS,
    scratch_types=[
        pltpu.VMEM((CH,), _i32),
        pltpu.VMEM((CH,), _i32),
        pltpu.VMEM((4 * CH,), _f32),
        pltpu.VMEM((4 * CH,), _f32),
        pltpu.VMEM((4 * CH,), _f32),
        pltpu.VMEM((4 * CH,), _f32),
        pltpu.VMEM((2 * CH,), _f32),
        pltpu.VMEM((2 * CH,), _f32),
        pltpu.VMEM((64,), _f32),
        pltpu.SemaphoreType.DMA,
        pltpu.SemaphoreType.DMA,
        pltpu.SemaphoreType.DMA,
        [pltpu.VMEM_SHARED((NPAD,), _f32) for _ in range(4)],
        [pltpu.VMEM_SHARED((NPAD,), _f32) for _ in range(4)],
        [pltpu.VMEM_SHARED((NPAD,), _f32) for _ in range(4)],
    ],
)
def _k1a(as_hbm, ad_hbm, c1_hbm, src_hbm, dst_hbm, coef_hbm, e_hbm,
         srcc, dstc, asb, adb, ebuf, dbuf, cbufa, cbufb, c1v,
         lsem, gsem, wsem, sh_as, sh_ad, sh_den):
    c = lax.axis_index("c")
    s = lax.axis_index("s")

    pltpu.sync_copy(c1_hbm.at[c], c1v)
    for h in range(4):
        pltpu.sync_copy(as_hbm.at[c, h, pl.ds(s * NPT, NPT)],
                        sh_as[h].at[pl.ds(s * NPT, NPT)])
        pltpu.sync_copy(ad_hbm.at[c, h, pl.ds(s * NPT, NPT)],
                        sh_ad[h].at[pl.ds(s * NPT, NPT)])

    iota = lax.iota(_i32, 16)
    z16 = jnp.zeros((16,), _f32)
    for g in range(8):
        ebuf[pl.ds(16 * g, 16)] = z16
    for h in range(4):
        for i in range(5):
            pltpu.sync_copy(ebuf.at[pl.ds(0, CH)],
                            sh_den[h].at[pl.ds(s * NPT + i * CH, CH)])

    plsc.subcore_barrier()

    c1h = [c1v[pl.ds(16 * h, 16)] for h in range(4)]

    @pl.loop(0, CHUNKS)
    def _pass1(k):
        pltpu.async_copy(src_hbm.at[s, k], srcc, lsem)
        pltpu.async_copy(dst_hbm.at[s, k], dstc, lsem)
        @pl.when(k > 0)
        def _():
            pltpu.make_async_copy(ebuf, e_hbm.at[c, s, k - 1], wsem).wait()
        pltpu.make_async_copy(src_hbm.at[s, k], srcc, lsem).wait()
        pltpu.make_async_copy(dst_hbm.at[s, k], dstc, lsem).wait()
        for h in range(4):
            pltpu.async_copy(sh_as[h].at[srcc], asb.at[pl.ds(CH * h, CH)],
                             gsem)
            pltpu.async_copy(sh_ad[h].at[dstc], adb.at[pl.ds(CH * h, CH)],
                             gsem)
        for h in range(4):
            pltpu.make_async_copy(sh_as[h].at[srcc],
                                  asb.at[pl.ds(CH * h, CH)], gsem).wait()
            pltpu.make_async_copy(sh_ad[h].at[dstc],
                                  adb.at[pl.ds(CH * h, CH)], gsem).wait()
        base = s * EPT + k * CH
        for g in range(8):
            mask = (base + 16 * g + iota) < NE
            for h in range(4):
                o = CH * h + 16 * g
                a = asb[pl.ds(o, 16)] + adb[pl.ds(o, 16)]
                a = jnp.maximum(a, 0.2 * a)
                e = jnp.where(mask, jnp.exp(a - c1h[h]), 0.0)
                ebuf[pl.ds(o, 16)] = e
        pltpu.async_copy(ebuf, e_hbm.at[c, s, k], wsem)
        for h in range(4):
            pltpu.sync_copy(ebuf.at[pl.ds(CH * h, CH)],
                            sh_den[h].at[dstc], add=True)

    pltpu.make_async_copy(ebuf, e_hbm.at[c, s, CHUNKS - 1], wsem).wait()
    plsc.subcore_barrier()

    @pl.loop(0, CHUNKS)
    def _pass2(k):
        pltpu.async_copy(dst_hbm.at[s, k], dstc, lsem)
        pltpu.async_copy(e_hbm.at[c, s, k], ebuf, lsem)
        @pl.when(k > 0)
        def _():
            pltpu.make_async_copy(cbufa, coef_hbm.at[c, 0, s, k - 1],
                                  wsem).wait()
            pltpu.make_async_copy(cbufb, coef_hbm.at[c, 1, s, k - 1],
                                  wsem).wait()
        pltpu.make_async_copy(dst_hbm.at[s, k], dstc, lsem).wait()
        pltpu.make_async_copy(e_hbm.at[c, s, k], ebuf, lsem).wait()
        for h in range(4):
            pltpu.async_copy(sh_den[h].at[dstc], dbuf.at[pl.ds(CH * h, CH)],
                             gsem)
        for h in range(4):
            pltpu.make_async_copy(sh_den[h].at[dstc],
                                  dbuf.at[pl.ds(CH * h, CH)], gsem).wait()
        for p in range(2):
            cb = cbufa if p == 0 else cbufb
            for j in range(2):
                h = 2 * p + j
                for g in range(8):
                    o = CH * h + 16 * g
                    ev = ebuf[pl.ds(o, 16)]
                    dv = dbuf[pl.ds(o, 16)]
                    cb[pl.ds(CH * j + 16 * g, 16)] = ev / (dv + 1e-16)
        pltpu.async_copy(cbufa, coef_hbm.at[c, 0, s, k], wsem)
        pltpu.async_copy(cbufb, coef_hbm.at[c, 1, s, k], wsem)

    pltpu.make_async_copy(cbufa, coef_hbm.at[c, 0, s, CHUNKS - 1],
                          wsem).wait()
    pltpu.make_async_copy(cbufb, coef_hbm.at[c, 1, s, CHUNKS - 1],
                          wsem).wait()


def _make_k1b(p):
    @functools.partial(
        pl.kernel,
        out_type=jax.ShapeDtypeStruct((2, NPAD, 64), _f32),
        mesh=_MESH,
        compiler_params=_SC_PARAMS,
        scratch_types=[
            pltpu.VMEM((CH,), _i32),
            pltpu.VMEM((CH,), _i32),
            pltpu.VMEM((CH,), _i32),
            pltpu.VMEM((CH,), _i32),
            pltpu.VMEM((2 * CH,), _f32),
            pltpu.VMEM((2 * CH,), _f32),
            pltpu.VMEM((CH, 64), _f32),
            pltpu.VMEM((CH, 64), _f32),
            pltpu.SemaphoreType.DMA,
            pltpu.SemaphoreType.DMA,
            pltpu.SemaphoreType.DMA,
            pltpu.SemaphoreType.DMA,
            pltpu.VMEM_SHARED((NPAD, 64), _f32),
        ],
    )
    def _k1b(h_hbm, coef_hbm, adj_hbm, dst_hbm, out_hbm,
             adjc0, adjc1, dstc0, dstc1, cbuf0, cbuf1, hbuf0, hbuf1,
             gsem0, gsem1, ssem0, ssem1, sh_out):
        c = lax.axis_index("c")
        s = lax.axis_index("s")
        adjc = (adjc0, adjc1)
        dstc = (dstc0, dstc1)
        cbuf = (cbuf0, cbuf1)
        hbuf = (hbuf0, hbuf1)
        gsem = (gsem0, gsem1)
        ssem = (ssem0, ssem1)

        z16 = jnp.zeros((16,), _f32)

        @pl.loop(0, CH)
        def _zrow(r):
            for u in range(4):
                hbuf0[r, pl.ds(16 * u, 16)] = z16

        for i in range(5):
            pltpu.sync_copy(hbuf0, sh_out.at[pl.ds(s * NPT + i * CH, CH)])

        plsc.subcore_barrier()

        def load_and_gather(b, k):
            pltpu.sync_copy(adj_hbm.at[c, s, k], adjc[b])
            pltpu.sync_copy(dst_hbm.at[s, k], dstc[b])
            pltpu.sync_copy(coef_hbm.at[c, p, s, k], cbuf[b])
            pltpu.async_copy(h_hbm.at[adjc[b]], hbuf[b], gsem[b])

        def mult(b):
            @pl.loop(0, CH)
            def _mul(e):
                for j in range(2):
                    cs = plsc.load_gather(
                        cbuf[b], [jnp.full((16,), CH * j + e, _i32)])
                    for u in range(2):
                        off = 32 * j + 16 * u
                        hbuf[b][e, pl.ds(off, 16)] = (
                            hbuf[b][e, pl.ds(off, 16)] * cs)

        load_and_gather(0, 0)
        load_and_gather(1, 1)

        NH = CHUNKS // 2

        @pl.loop(0, NH)
        def _msg(t):
            for b in range(2):
                pltpu.make_async_copy(h_hbm.at[adjc[b]], hbuf[b],
                                      gsem[b]).wait()
                mult(b)
                pltpu.async_copy(hbuf[b], sh_out.at[dstc[b]], ssem[b],
                                 add=True)

            @pl.when(t + 1 < NH)
            def _():
                for b in range(2):
                    pltpu.make_async_copy(hbuf[b], sh_out.at[dstc[b]],
                                          ssem[b]).wait()
                    load_and_gather(b, 2 * t + 2 + b)

        for b in range(2):
            pltpu.make_async_copy(hbuf[b], sh_out.at[dstc[b]],
                                  ssem[b]).wait()

        plsc.subcore_barrier()
        for i in range(5):
            pltpu.sync_copy(sh_out.at[pl.ds(s * NPT + i * CH, CH)], hbuf0)
            pltpu.sync_copy(hbuf0, out_hbm.at[c, pl.ds(s * NPT + i * CH, CH)])

    return _k1b


_K1B = (_make_k1b(0), _make_k1b(1))


def _tc2_body(o1a_ref, o1b_ref, b1_ref, w2_ref, sd2_ref, h2_ref, asd_ref,
              cm_ref):
    r = pl.program_id(0)
    h2 = jnp.zeros((BR, DIM), _f32)
    for i, (ref, ci) in enumerate(
            ((o1a_ref, 0), (o1b_ref, 0), (o1a_ref, 1), (o1b_ref, 1))):
        x1 = ref[ci] + b1_ref[i][None]
        x1 = jnp.where(x1 > 0, x1, jnp.exp(jnp.minimum(x1, 0.0)) - 1.0)
        h2 = h2 + jnp.dot(x1, w2_ref[i], preferred_element_type=_f32)
    h2_ref[...] = h2
    asd = jnp.dot(h2, sd2_ref[...], preferred_element_type=_f32)
    asd_ref[...] = asd

    @pl.when(r == 0)
    def _():
        cm_ref[...] = jnp.full((1, 1, 2), -jnp.inf, _f32)

    cm_ref[...] = jnp.maximum(cm_ref[...], asd.max(axis=0)[None, None])


def _tc2(o1a, o1b, b1q, w2q, sd2):
    return pl.pallas_call(
        _tc2_body,
        grid=(NB,),
        in_specs=[
            pl.BlockSpec((2, BR, 64), lambda r: (0, r, 0)),
            pl.BlockSpec((2, BR, 64), lambda r: (0, r, 0)),
            pl.BlockSpec((4, 64), lambda r: (0, 0)),
            pl.BlockSpec((4, 64, DIM), lambda r: (0, 0, 0)),
            pl.BlockSpec((DIM, 2), lambda r: (0, 0)),
        ],
        out_specs=[
            pl.BlockSpec((BR, DIM), lambda r: (r, 0)),
            pl.BlockSpec((BR, 2), lambda r: (r, 0)),
            pl.BlockSpec((1, 1, 2), lambda r: (0, 0, 0)),
        ],
        out_shape=[
            jax.ShapeDtypeStruct((N, DIM), _f32),
            jax.ShapeDtypeStruct((N, 2), _f32),
            jax.ShapeDtypeStruct((1, 1, 2), _f32),
        ],
    )(o1a, o1b, b1q, w2q, sd2)


@functools.partial(
    pl.kernel,
    out_type=jax.ShapeDtypeStruct((NPAD,), _f32),
    mesh=_MESH,
    compiler_params=_SC_PARAMS,
    scratch_types=[
        pltpu.VMEM((N,), _f32),
        pltpu.VMEM((N,), _f32),
        pltpu.VMEM((NPAD,), _f32),
        pltpu.VMEM((CHUNKS, CH), _f32),
        pltpu.VMEM((CH,), _i32),
        pltpu.VMEM((CH,), _i32),
        pltpu.VMEM((CH,), _f32),
        pltpu.VMEM((16,), _f32),
        pltpu.VMEM_SHARED((NPAD,), _f32),
        pltpu.VMEM_SHARED((NPAD,), _f32),
    ],
)
def _k2(as2_hbm, ad2_hbm, c2_hbm, src_hbm, dst_hbm, c_hbm,
        as2fl, ad2fl, d2b, e2buf, srcc, dstc, cbuf, c2v, sh_d2, sh_c):
    c = lax.axis_index("c")
    s = lax.axis_index("s")

    @pl.when(c == 0)
    def _():
        pltpu.sync_copy(as2_hbm, as2fl)
        pltpu.sync_copy(ad2_hbm, ad2fl)
        pltpu.sync_copy(c2_hbm, c2v)

        z16 = jnp.zeros((16,), _f32)

        @pl.loop(0, CH // 16)
        def _zr1(r):
            d2b[pl.ds(16 * r, 16)] = z16

        for i in range(5):
            pltpu.sync_copy(d2b.at[pl.ds(0, CH)],
                            sh_d2.at[pl.ds(s * NPT + i * CH, CH)])
            pltpu.sync_copy(d2b.at[pl.ds(0, CH)],
                            sh_c.at[pl.ds(s * NPT + i * CH, CH)])

        plsc.subcore_barrier()

        iota = lax.iota(_i32, 16)
        c2t = c2v[...]

        @pl.loop(0, CHUNKS)
        def _pass1(k):
            pltpu.sync_copy(src_hbm.at[s, k], srcc)
            pltpu.sync_copy(dst_hbm.at[s, k], dstc)
            base = s * EPT + k * CH
            for g in range(8):
                sv = srcc[pl.ds(16 * g, 16)]
                dv = dstc[pl.ds(16 * g, 16)]
                a = (plsc.load_gather(as2fl, [sv])
                     + plsc.load_gather(ad2fl, [dv]))
                a = jnp.maximum(a, 0.2 * a)
                e = jnp.where((base + 16 * g + iota) < NE,
                              jnp.exp(a - c2t), 0.0)
                e2buf[k, pl.ds(16 * g, 16)] = e
            pltpu.sync_copy(e2buf.at[k], sh_d2.at[dstc], add=True)

        plsc.subcore_barrier()
        pltpu.sync_copy(sh_d2, d2b)

        @pl.loop(0, CHUNKS)
        def _pass2(k):
            pltpu.sync_copy(src_hbm.at[s, k], srcc)
            pltpu.sync_copy(dst_hbm.at[s, k], dstc)
            for g in range(8):
                ev = e2buf[k, pl.ds(16 * g, 16)]
                dv2 = plsc.load_gather(d2b, [dstc[pl.ds(16 * g, 16)]])
                cbuf[pl.ds(16 * g, 16)] = ev / (dv2 + 1e-16)
            pltpu.sync_copy(cbuf, sh_c.at[srcc], add=True)

        plsc.subcore_barrier()
        pltpu.sync_copy(sh_c.at[pl.ds(s * NPT, NPT)],
                        c_hbm.at[pl.ds(s * NPT, NPT)])


def _tc3_body(c_ref, h2_ref, b2_ref, l1w_ref, l1b_ref, l2w_ref, l2b_ref,
              out_ref, acc_ref):
    r = pl.program_id(0)

    @pl.when(r == 0)
    def _():
        acc_ref[...] = jnp.zeros((1, DIM), _f32)

    acc_ref[...] += jnp.dot(c_ref[0], h2_ref[...],
                            preferred_element_type=_f32)

    @pl.when(r == NB - 1)
    def _():
        g = acc_ref[...] + float(N) * b2_ref[...]
        g1 = jnp.maximum(
            jnp.dot(g, l1w_ref[...], preferred_element_type=_f32)
            + l1b_ref[...], 0.0)
        out_ref[...] = (jnp.dot(g1, l2w_ref[...], preferred_element_type=_f32)
                        + l2b_ref[...])


def _tc3(c2d, h2, b2r, l1w, l1b, l2w, l2b):
    return pl.pallas_call(
        _tc3_body,
        grid=(NB,),
        in_specs=[
            pl.BlockSpec((1, 1, BR), lambda r: (r, 0, 0)),
            pl.BlockSpec((BR, DIM), lambda r: (r, 0)),
            pl.BlockSpec((1, DIM), lambda r: (0, 0)),
            pl.BlockSpec((DIM, DIM), lambda r: (0, 0)),
            pl.BlockSpec((1, DIM), lambda r: (0, 0)),
            pl.BlockSpec((DIM, OUT), lambda r: (0, 0)),
            pl.BlockSpec((1, OUT), lambda r: (0, 0)),
        ],
        out_specs=pl.BlockSpec((1, OUT), lambda r: (0, 0)),
        out_shape=jax.ShapeDtypeStruct((1, OUT), _f32),
        scratch_shapes=[pltpu.VMEM((1, DIM), _f32)],
    )(c2d, h2, b2r, l1w, l1b, l2w, l2b)


def _blockdiag2(att):
    z = jnp.zeros((2, DIM, 2), att.dtype)
    z = z.at[jnp.arange(2), :, jnp.arange(2)].set(att)
    return z.reshape(2 * DIM, 2)


def kernel(x, edge_index, W1, att_s1, att_d1, b1, W2, att_s2, att_d2, b2,
           lin1_W, lin1_b, lin2_W, lin2_b):
    loops = jnp.arange(N, dtype=_i32)
    pad = jnp.zeros((E_PAD - NE,), _i32)
    src = jnp.concatenate([edge_index[0].astype(_i32), loops, pad])
    dst = jnp.concatenate([edge_index[1].astype(_i32), loops, pad])
    src3 = src.reshape(16, CHUNKS, CH)
    dst3 = dst.reshape(16, CHUNKS, CH)
    adj4 = src3[None] + (jnp.arange(4, dtype=_i32) * N)[:, None, None, None]

    w1q = W1.reshape(IN, 4, 64).transpose(1, 0, 2)
    sa = jnp.stack([_blockdiag2(att_s1[2 * q:2 * q + 2]) for q in range(4)])
    sd = jnp.stack([_blockdiag2(att_d1[2 * q:2 * q + 2]) for q in range(4)])

    h4, as4, ad4, cs, cd = _tc1(x.astype(_f32), w1q, sa, sd)
    as_p = jnp.pad(as4.transpose(0, 2, 1).reshape(2, 4, N),
                   ((0, 0), (0, 0), (0, NPAD - N)))
    ad_p = jnp.pad(ad4.transpose(0, 2, 1).reshape(2, 4, N),
                   ((0, 0), (0, 0), (0, NPAD - N)))
    c1 = (cs + cd).reshape(2, 4)
    c1 = jnp.tile(c1[:, :, None], (1, 1, 16)).reshape(2, 64)

    coef, _unused_e = _k1a(as_p, ad_p, c1, src3, dst3)

    h_flat = h4.reshape(4 * N, 64)
    o1a = _K1B[0](h_flat, coef, adj4[0::2], dst3)
    o1b = _K1B[1](h_flat, coef, adj4[1::2], dst3)

    sd2 = jnp.stack([att_s2[0], att_d2[0]], axis=1)
    b1q = b1.reshape(4, 64)
    w2q = W2.reshape(4, 64, DIM)
    h2, asd2, cm2 = _tc2(o1a, o1b, b1q, w2q, sd2)
    c2arr = jnp.full((16,), cm2[0, 0, 0] + cm2[0, 0, 1], _f32)

    c_pad = _k2(asd2[:, 0], asd2[:, 1], c2arr, src3, dst3)

    c2d = c_pad[:N].reshape(NB, 1, BR)
    return _tc3(c2d, h2, b2.reshape(1, DIM), lin1_W, lin1_b.reshape(1, DIM),
                lin2_W, lin2_b.reshape(1, OUT))


# speedup vs baseline: 31.1887x; 1.1439x over previous
import functools

import jax
import jax.numpy as jnp
from jax import lax
from jax.experimental import pallas as pl
from jax.experimental.pallas import tpu as pltpu
from jax.experimental.pallas import tpu_sc as plsc

N = 10000
IN = 128
HEADS = 8
DIM = 32
OUT = 16

NE = 330000
CH = 128
CHUNKS = 162
EPT = CHUNKS * CH
E_PAD = 16 * EPT
NPAD = 10240
NPT = 640
BR = 1000
NB = N // BR

_f32 = jnp.float32
_i32 = jnp.int32

_MESH = plsc.VectorSubcoreMesh(
    core_axis_name="c", subcore_axis_name="s", num_cores=2, num_subcores=16)
_SC_PARAMS = pltpu.CompilerParams(needs_layout_passes=False,
                                  use_tc_tiling_on_sc=False)


def _tc1_body(x_ref, w_ref, sa_ref, sd_ref, h_ref, as_ref, ad_ref, cs_ref,
              cd_ref):
    r = pl.program_id(1)
    h = jnp.dot(x_ref[...], w_ref[0], preferred_element_type=_f32)
    h_ref[...] = h[None]
    a_s = jnp.dot(h, sa_ref[0], preferred_element_type=_f32)
    a_d = jnp.dot(h, sd_ref[0], preferred_element_type=_f32)
    as_ref[...] = a_s[None]
    ad_ref[...] = a_d[None]

    @pl.when(r == 0)
    def _():
        cs_ref[...] = jnp.full((1, 1, 2), -jnp.inf, _f32)
        cd_ref[...] = jnp.full((1, 1, 2), -jnp.inf, _f32)

    cs_ref[...] = jnp.maximum(cs_ref[...], a_s.max(axis=0)[None, None])
    cd_ref[...] = jnp.maximum(cd_ref[...], a_d.max(axis=0)[None, None])


def _tc1(x, w1q, sa, sd):
    return pl.pallas_call(
        _tc1_body,
        grid=(4, NB),
        in_specs=[
            pl.BlockSpec((BR, IN), lambda q, r: (r, 0)),
            pl.BlockSpec((1, IN, 64), lambda q, r: (q, 0, 0)),
            pl.BlockSpec((1, 64, 2), lambda q, r: (q, 0, 0)),
            pl.BlockSpec((1, 64, 2), lambda q, r: (q, 0, 0)),
        ],
        out_specs=[
            pl.BlockSpec((1, BR, 64), lambda q, r: (q, r, 0)),
            pl.BlockSpec((1, BR, 2), lambda q, r: (q, r, 0)),
            pl.BlockSpec((1, BR, 2), lambda q, r: (q, r, 0)),
            pl.BlockSpec((1, 1, 2), lambda q, r: (q, 0, 0)),
            pl.BlockSpec((1, 1, 2), lambda q, r: (q, 0, 0)),
        ],
        out_shape=[
            jax.ShapeDtypeStruct((4, N, 64), _f32),
            jax.ShapeDtypeStruct((4, N, 2), _f32),
            jax.ShapeDtypeStruct((4, N, 2), _f32),
            jax.ShapeDtypeStruct((4, 1, 2), _f32),
            jax.ShapeDtypeStruct((4, 1, 2), _f32),
        ],
    )(x, w1q, sa, sd)


@functools.partial(
    pl.kernel,
    out_type=(
        jax.ShapeDtypeStruct((2, 2, 16, CHUNKS, 2 * CH), _f32),
        jax.ShapeDtypeStruct((2, 16, CHUNKS, 4 * CH), _f32),
    ),
    mesh=_MESH,
    compiler_params=_SC_PARAMS,
    scratch_types=[
        pltpu.VMEM((CH,), _i32),
        pltpu.VMEM((CH,), _i32),
        pltpu.VMEM((4 * CH,), _f32),
        pltpu.VMEM((4 * CH,), _f32),
        pltpu.VMEM((4 * CH,), _f32),
        pltpu.VMEM((4 * CH,), _f32),
        pltpu.VMEM((2 * CH,), _f32),
        pltpu.VMEM((2 * CH,), _f32),
        pltpu.VMEM((64,), _f32),
        pltpu.SemaphoreType.DMA,
        pltpu.SemaphoreType.DMA,
        pltpu.SemaphoreType.DMA,
        [pltpu.VMEM_SHARED((NPAD,), _f32) for _ in range(4)],
        [pltpu.VMEM_SHARED((NPAD,), _f32) for _ in range(4)],
        [pltpu.VMEM_SHARED((NPAD,), _f32) for _ in range(4)],
    ],
)
def _k1a(as_hbm, ad_hbm, c1_hbm, src_hbm, dst_hbm, coef_hbm, e_hbm,
         srcc, dstc, asb, adb, ebuf, dbuf, cbufa, cbufb, c1v,
         lsem, gsem, wsem, sh_as, sh_ad, sh_den):
    c = lax.axis_index("c")
    s = lax.axis_index("s")

    pltpu.sync_copy(c1_hbm.at[c], c1v)
    for h in range(4):
        pltpu.sync_copy(as_hbm.at[c, h, pl.ds(s * NPT, NPT)],
                        sh_as[h].at[pl.ds(s * NPT, NPT)])
        pltpu.sync_copy(ad_hbm.at[c, h, pl.ds(s * NPT, NPT)],
                        sh_ad[h].at[pl.ds(s * NPT, NPT)])

    iota = lax.iota(_i32, 16)
    z16 = jnp.zeros((16,), _f32)
    for g in range(8):
        ebuf[pl.ds(16 * g, 16)] = z16
    for h in range(4):
        for i in range(5):
            pltpu.sync_copy(ebuf.at[pl.ds(0, CH)],
                            sh_den[h].at[pl.ds(s * NPT + i * CH, CH)])

    plsc.subcore_barrier()

    c1h = [c1v[pl.ds(16 * h, 16)] for h in range(4)]

    @pl.loop(0, CHUNKS)
    def _pass1(k):
        pltpu.async_copy(src_hbm.at[s, k], srcc, lsem)
        pltpu.async_copy(dst_hbm.at[s, k], dstc, lsem)
        @pl.when(k > 0)
        def _():
            pltpu.make_async_copy(ebuf, e_hbm.at[c, s, k - 1], wsem).wait()
        pltpu.make_async_copy(src_hbm.at[s, k], srcc, lsem).wait()
        pltpu.make_async_copy(dst_hbm.at[s, k], dstc, lsem).wait()
        for h in range(4):
            pltpu.async_copy(sh_as[h].at[srcc], asb.at[pl.ds(CH * h, CH)],
                             gsem)
            pltpu.async_copy(sh_ad[h].at[dstc], adb.at[pl.ds(CH * h, CH)],
                             gsem)
        for h in range(4):
            pltpu.make_async_copy(sh_as[h].at[srcc],
                                  asb.at[pl.ds(CH * h, CH)], gsem).wait()
            pltpu.make_async_copy(sh_ad[h].at[dstc],
                                  adb.at[pl.ds(CH * h, CH)], gsem).wait()
        base = s * EPT + k * CH
        for g in range(8):
            mask = (base + 16 * g + iota) < NE
            for h in range(4):
                o = CH * h + 16 * g
                a = asb[pl.ds(o, 16)] + adb[pl.ds(o, 16)]
                a = jnp.maximum(a, 0.2 * a)
                e = jnp.where(mask, jnp.exp(a - c1h[h]), 0.0)
                ebuf[pl.ds(o, 16)] = e
        pltpu.async_copy(ebuf, e_hbm.at[c, s, k], wsem)
        for h in range(4):
            pltpu.sync_copy(ebuf.at[pl.ds(CH * h, CH)],
                            sh_den[h].at[dstc], add=True)

    pltpu.make_async_copy(ebuf, e_hbm.at[c, s, CHUNKS - 1], wsem).wait()
    plsc.subcore_barrier()

    @pl.loop(0, CHUNKS)
    def _pass2(k):
        pltpu.async_copy(dst_hbm.at[s, k], dstc, lsem)
        pltpu.async_copy(e_hbm.at[c, s, k], ebuf, lsem)
        @pl.when(k > 0)
        def _():
            pltpu.make_async_copy(cbufa, coef_hbm.at[c, 0, s, k - 1],
                                  wsem).wait()
            pltpu.make_async_copy(cbufb, coef_hbm.at[c, 1, s, k - 1],
                                  wsem).wait()
        pltpu.make_async_copy(dst_hbm.at[s, k], dstc, lsem).wait()
        pltpu.make_async_copy(e_hbm.at[c, s, k], ebuf, lsem).wait()
        for h in range(4):
            pltpu.async_copy(sh_den[h].at[dstc], dbuf.at[pl.ds(CH * h, CH)],
                             gsem)
        for h in range(4):
            pltpu.make_async_copy(sh_den[h].at[dstc],
                                  dbuf.at[pl.ds(CH * h, CH)], gsem).wait()
        for p in range(2):
            cb = cbufa if p == 0 else cbufb
            for j in range(2):
                h = 2 * p + j
                for g in range(8):
                    o = CH * h + 16 * g
                    ev = ebuf[pl.ds(o, 16)]
                    dv = dbuf[pl.ds(o, 16)]
                    cb[pl.ds(CH * j + 16 * g, 16)] = ev / (dv + 1e-16)
        pltpu.async_copy(cbufa, coef_hbm.at[c, 0, s, k], wsem)
        pltpu.async_copy(cbufb, coef_hbm.at[c, 1, s, k], wsem)

    pltpu.make_async_copy(cbufa, coef_hbm.at[c, 0, s, CHUNKS - 1],
                          wsem).wait()
    pltpu.make_async_copy(cbufb, coef_hbm.at[c, 1, s, CHUNKS - 1],
                          wsem).wait()


def _make_k1b(p):
    @functools.partial(
        pl.kernel,
        out_type=jax.ShapeDtypeStruct((2, NPAD, 64), _f32),
        mesh=_MESH,
        compiler_params=_SC_PARAMS,
        scratch_types=[
            pltpu.VMEM((CH,), _i32),
            pltpu.VMEM((CH,), _i32),
            pltpu.VMEM((CH,), _i32),
            pltpu.VMEM((CH,), _i32),
            pltpu.VMEM((2 * CH,), _f32),
            pltpu.VMEM((2 * CH,), _f32),
            pltpu.VMEM((CH, 64), _f32),
            pltpu.VMEM((CH, 64), _f32),
            pltpu.SemaphoreType.DMA,
            pltpu.SemaphoreType.DMA,
            pltpu.SemaphoreType.DMA,
            pltpu.SemaphoreType.DMA,
            pltpu.VMEM_SHARED((NPAD, 64), _f32),
        ],
    )
    def _k1b(h_hbm, coef_hbm, adj_hbm, dst_hbm, out_hbm,
             adjc0, adjc1, dstc0, dstc1, cbuf0, cbuf1, hbuf0, hbuf1,
             gsem0, gsem1, ssem0, ssem1, sh_out):
        c = lax.axis_index("c")
        s = lax.axis_index("s")
        adjc = (adjc0, adjc1)
        dstc = (dstc0, dstc1)
        cbuf = (cbuf0, cbuf1)
        hbuf = (hbuf0, hbuf1)
        gsem = (gsem0, gsem1)
        ssem = (ssem0, ssem1)

        z16 = jnp.zeros((16,), _f32)

        @pl.loop(0, CH)
        def _zrow(r):
            for u in range(4):
                hbuf0[r, pl.ds(16 * u, 16)] = z16

        for i in range(5):
            pltpu.sync_copy(hbuf0, sh_out.at[pl.ds(s * NPT + i * CH, CH)])

        plsc.subcore_barrier()

        def load_and_gather(b, k):
            pltpu.sync_copy(adj_hbm.at[c, s, k], adjc[b])
            pltpu.sync_copy(dst_hbm.at[s, k], dstc[b])
            pltpu.sync_copy(coef_hbm.at[c, p, s, k], cbuf[b])
            pltpu.async_copy(h_hbm.at[adjc[b]], hbuf[b], gsem[b])

        def mult(b):
            @pl.loop(0, CH)
            def _mul(e):
                for j in range(2):
                    cs = plsc.load_gather(
                        cbuf[b], [jnp.full((16,), CH * j + e, _i32)])
                    for u in range(2):
                        off = 32 * j + 16 * u
                        hbuf[b][e, pl.ds(off, 16)] = (
                            hbuf[b][e, pl.ds(off, 16)] * cs)

        load_and_gather(0, 0)
        load_and_gather(1, 1)

        NH = CHUNKS // 2

        @pl.loop(0, NH)
        def _msg(t):
            for b in range(2):
                pltpu.make_async_copy(h_hbm.at[adjc[b]], hbuf[b],
                                      gsem[b]).wait()
                mult(b)
                pltpu.async_copy(hbuf[b], sh_out.at[dstc[b]], ssem[b],
                                 add=True)

            @pl.when(t + 1 < NH)
            def _():
                for b in range(2):
                    pltpu.make_async_copy(hbuf[b], sh_out.at[dstc[b]],
                                          ssem[b]).wait()
                    load_and_gather(b, 2 * t + 2 + b)

        for b in range(2):
            pltpu.make_async_copy(hbuf[b], sh_out.at[dstc[b]],
                                  ssem[b]).wait()

        plsc.subcore_barrier()
        for i in range(5):
            pltpu.sync_copy(sh_out.at[pl.ds(s * NPT + i * CH, CH)], hbuf0)
            pltpu.sync_copy(hbuf0, out_hbm.at[c, pl.ds(s * NPT + i * CH, CH)])

    return _k1b


_K1B = (_make_k1b(0), _make_k1b(1))


def _tc2_body(o1a_ref, o1b_ref, b1_ref, w2_ref, sd2_ref, h2_ref, asd_ref,
              cm_ref):
    r = pl.program_id(0)
    h2 = jnp.zeros((BR, DIM), _f32)
    for i, (ref, ci) in enumerate(
            ((o1a_ref, 0), (o1b_ref, 0), (o1a_ref, 1), (o1b_ref, 1))):
        x1 = ref[ci] + b1_ref[i][None]
        x1 = jnp.where(x1 > 0, x1, jnp.exp(jnp.minimum(x1, 0.0)) - 1.0)
        h2 = h2 + jnp.dot(x1, w2_ref[i], preferred_element_type=_f32)
    h2_ref[...] = h2
    asd = jnp.dot(h2, sd2_ref[...], preferred_element_type=_f32)
    asd_ref[...] = asd

    @pl.when(r == 0)
    def _():
        cm_ref[...] = jnp.full((1, 1, 2), -jnp.inf, _f32)

    cm_ref[...] = jnp.maximum(cm_ref[...], asd.max(axis=0)[None, None])


def _tc2(o1a, o1b, b1q, w2q, sd2):
    return pl.pallas_call(
        _tc2_body,
        grid=(NB,),
        in_specs=[
            pl.BlockSpec((2, BR, 64), lambda r: (0, r, 0)),
            pl.BlockSpec((2, BR, 64), lambda r: (0, r, 0)),
            pl.BlockSpec((4, 64), lambda r: (0, 0)),
            pl.BlockSpec((4, 64, DIM), lambda r: (0, 0, 0)),
            pl.BlockSpec((DIM, 2), lambda r: (0, 0)),
        ],
        out_specs=[
            pl.BlockSpec((BR, DIM), lambda r: (r, 0)),
            pl.BlockSpec((BR, 2), lambda r: (r, 0)),
            pl.BlockSpec((1, 1, 2), lambda r: (0, 0, 0)),
        ],
        out_shape=[
            jax.ShapeDtypeStruct((N, DIM), _f32),
            jax.ShapeDtypeStruct((N, 2), _f32),
            jax.ShapeDtypeStruct((1, 1, 2), _f32),
        ],
    )(o1a, o1b, b1q, w2q, sd2)


@functools.partial(
    pl.kernel,
    out_type=jax.ShapeDtypeStruct((NPAD,), _f32),
    mesh=_MESH,
    compiler_params=_SC_PARAMS,
    scratch_types=[
        pltpu.VMEM((N,), _f32),
        pltpu.VMEM((N,), _f32),
        pltpu.VMEM((NPAD,), _f32),
        pltpu.VMEM((CHUNKS, CH), _f32),
        [pltpu.VMEM((CH,), _i32) for _ in range(2)],
        [pltpu.VMEM((CH,), _i32) for _ in range(2)],
        [pltpu.VMEM((CH,), _f32) for _ in range(2)],
        pltpu.VMEM((16,), _f32),
        pltpu.SemaphoreType.DMA,
        pltpu.SemaphoreType.DMA,
        pltpu.SemaphoreType.DMA,
        pltpu.VMEM_SHARED((NPAD,), _f32),
        pltpu.VMEM_SHARED((NPAD,), _f32),
    ],
)
def _k2(as2_hbm, ad2_hbm, c2_hbm, src_hbm, dst_hbm, c_hbm,
        as2fl, ad2fl, d2b, e2buf, srcc, dstc, cbuf, c2v,
        lsem, wsem0, wsem1, sh_d2, sh_c):
    wsem = (wsem0, wsem1)
    c = lax.axis_index("c")
    s = lax.axis_index("s")

    @pl.when(c == 0)
    def _():
        pltpu.sync_copy(as2_hbm, as2fl)
        pltpu.sync_copy(ad2_hbm, ad2fl)
        pltpu.sync_copy(c2_hbm, c2v)

        z16 = jnp.zeros((16,), _f32)

        @pl.loop(0, CH // 16)
        def _zr1(r):
            d2b[pl.ds(16 * r, 16)] = z16

        for i in range(5):
            pltpu.sync_copy(d2b.at[pl.ds(0, CH)],
                            sh_d2.at[pl.ds(s * NPT + i * CH, CH)])
            pltpu.sync_copy(d2b.at[pl.ds(0, CH)],
                            sh_c.at[pl.ds(s * NPT + i * CH, CH)])

        plsc.subcore_barrier()

        iota = lax.iota(_i32, 16)
        c2t = c2v[...]
        NH = CHUNKS // 2

        def idx_load(b, k):
            pltpu.async_copy(src_hbm.at[s, k], srcc[b], lsem)
            pltpu.async_copy(dst_hbm.at[s, k], dstc[b], lsem)

        def idx_drain(b, k):
            pltpu.make_async_copy(src_hbm.at[s, k], srcc[b], lsem).wait()
            pltpu.make_async_copy(dst_hbm.at[s, k], dstc[b], lsem).wait()

        idx_load(0, 0)
        idx_load(1, 1)

        @pl.loop(0, NH)
        def _pass1(t):
            for b in range(2):
                k = 2 * t + b
                idx_drain(b, k)
                base = s * EPT + k * CH
                for g in range(8):
                    sv = srcc[b][pl.ds(16 * g, 16)]
                    dv = dstc[b][pl.ds(16 * g, 16)]
                    a = (plsc.load_gather(as2fl, [sv])
                         + plsc.load_gather(ad2fl, [dv]))
                    a = jnp.maximum(a, 0.2 * a)
                    e = jnp.where((base + 16 * g + iota) < NE,
                                  jnp.exp(a - c2t), 0.0)
                    e2buf[k, pl.ds(16 * g, 16)] = e
                pltpu.async_copy(e2buf.at[k], sh_d2.at[dstc[b]], wsem[b],
                                 add=True)

            @pl.when(t + 1 < NH)
            def _():
                for b in range(2):
                    pltpu.make_async_copy(e2buf.at[2 * t + b],
                                          sh_d2.at[dstc[b]], wsem[b]).wait()
                    idx_load(b, 2 * t + 2 + b)

        for b in range(2):
            pltpu.make_async_copy(e2buf.at[CHUNKS - 2 + b],
                                  sh_d2.at[dstc[b]], wsem[b]).wait()

        plsc.subcore_barrier()
        pltpu.sync_copy(sh_d2, d2b)

        idx_load(0, 0)
        idx_load(1, 1)

        @pl.loop(0, NH)
        def _pass2(t):
            for b in range(2):
                k = 2 * t + b
                idx_drain(b, k)
                for g in range(8):
                    ev = e2buf[k, pl.ds(16 * g, 16)]
                    dv2 = plsc.load_gather(d2b, [dstc[b][pl.ds(16 * g, 16)]])
                    cbuf[b][pl.ds(16 * g, 16)] = ev / (dv2 + 1e-16)
                pltpu.async_copy(cbuf[b], sh_c.at[srcc[b]], wsem[b],
                                 add=True)

            @pl.when(t + 1 < NH)
            def _():
                for b in range(2):
                    pltpu.make_async_copy(cbuf[b], sh_c.at[srcc[b]],
                                          wsem[b]).wait()
                    idx_load(b, 2 * t + 2 + b)

        for b in range(2):
            pltpu.make_async_copy(cbuf[b], sh_c.at[srcc[b]], wsem[b]).wait()

        plsc.subcore_barrier()
        pltpu.sync_copy(sh_c.at[pl.ds(s * NPT, NPT)],
                        c_hbm.at[pl.ds(s * NPT, NPT)])


def _tc3_body(c_ref, h2_ref, b2_ref, l1w_ref, l1b_ref, l2w_ref, l2b_ref,
              out_ref, acc_ref):
    r = pl.program_id(0)

    @pl.when(r == 0)
    def _():
        acc_ref[...] = jnp.zeros((1, DIM), _f32)

    acc_ref[...] += jnp.dot(c_ref[0], h2_ref[...],
                            preferred_element_type=_f32)

    @pl.when(r == NB - 1)
    def _():
        g = acc_ref[...] + float(N) * b2_ref[...]
        g1 = jnp.maximum(
            jnp.dot(g, l1w_ref[...], preferred_element_type=_f32)
            + l1b_ref[...], 0.0)
        out_ref[...] = (jnp.dot(g1, l2w_ref[...], preferred_element_type=_f32)
                        + l2b_ref[...])


def _tc3(c2d, h2, b2r, l1w, l1b, l2w, l2b):
    return pl.pallas_call(
        _tc3_body,
        grid=(NB,),
        in_specs=[
            pl.BlockSpec((1, 1, BR), lambda r: (r, 0, 0)),
            pl.BlockSpec((BR, DIM), lambda r: (r, 0)),
            pl.BlockSpec((1, DIM), lambda r: (0, 0)),
            pl.BlockSpec((DIM, DIM), lambda r: (0, 0)),
            pl.BlockSpec((1, DIM), lambda r: (0, 0)),
            pl.BlockSpec((DIM, OUT), lambda r: (0, 0)),
            pl.BlockSpec((1, OUT), lambda r: (0, 0)),
        ],
        out_specs=pl.BlockSpec((1, OUT), lambda r: (0, 0)),
        out_shape=jax.ShapeDtypeStruct((1, OUT), _f32),
        scratch_shapes=[pltpu.VMEM((1, DIM), _f32)],
    )(c2d, h2, b2r, l1w, l1b, l2w, l2b)


def _blockdiag2(att):
    z = jnp.zeros((2, DIM, 2), att.dtype)
    z = z.at[jnp.arange(2), :, jnp.arange(2)].set(att)
    return z.reshape(2 * DIM, 2)


def kernel(x, edge_index, W1, att_s1, att_d1, b1, W2, att_s2, att_d2, b2,
           lin1_W, lin1_b, lin2_W, lin2_b):
    loops = jnp.arange(N, dtype=_i32)
    pad = jnp.zeros((E_PAD - NE,), _i32)
    src = jnp.concatenate([edge_index[0].astype(_i32), loops, pad])
    dst = jnp.concatenate([edge_index[1].astype(_i32), loops, pad])
    src3 = src.reshape(16, CHUNKS, CH)
    dst3 = dst.reshape(16, CHUNKS, CH)
    adj4 = src3[None] + (jnp.arange(4, dtype=_i32) * N)[:, None, None, None]

    w1q = W1.reshape(IN, 4, 64).transpose(1, 0, 2)
    sa = jnp.stack([_blockdiag2(att_s1[2 * q:2 * q + 2]) for q in range(4)])
    sd = jnp.stack([_blockdiag2(att_d1[2 * q:2 * q + 2]) for q in range(4)])

    h4, as4, ad4, cs, cd = _tc1(x.astype(_f32), w1q, sa, sd)
    as_p = jnp.pad(as4.transpose(0, 2, 1).reshape(2, 4, N),
                   ((0, 0), (0, 0), (0, NPAD - N)))
    ad_p = jnp.pad(ad4.transpose(0, 2, 1).reshape(2, 4, N),
                   ((0, 0), (0, 0), (0, NPAD - N)))
    c1 = (cs + cd).reshape(2, 4)
    c1 = jnp.tile(c1[:, :, None], (1, 1, 16)).reshape(2, 64)

    coef, _unused_e = _k1a(as_p, ad_p, c1, src3, dst3)

    h_flat = h4.reshape(4 * N, 64)
    o1a = _K1B[0](h_flat, coef, adj4[0::2], dst3)
    o1b = _K1B[1](h_flat, coef, adj4[1::2], dst3)

    sd2 = jnp.stack([att_s2[0], att_d2[0]], axis=1)
    b1q = b1.reshape(4, 64)
    w2q = W2.reshape(4, 64, DIM)
    h2, asd2, cm2 = _tc2(o1a, o1b, b1q, w2q, sd2)
    c2arr = jnp.full((16,), cm2[0, 0, 0] + cm2[0, 0, 1], _f32)

    c_pad = _k2(asd2[:, 0], asd2[:, 1], c2arr, src3, dst3)

    c2d = c_pad[:N].reshape(NB, 1, BR)
    return _tc3(c2d, h2, b2.reshape(1, DIM), lin1_W, lin1_b.reshape(1, DIM),
                lin2_W, lin2_b.reshape(1, OUT))
